# Initial kernel scaffold; baseline (speedup 1.0000x reference)
import functools

import jax
import jax.numpy as jnp
from jax import lax
from jax.experimental import pallas as pl
from jax.experimental.pallas import tpu as pltpu
from jax.experimental.pallas import tpu_sc as plsc

N = 10000
E = 320000
OUT_C = 64
HALF = 32
K = 10
ALPHA = 0.1
BN_EPS = 1e-5

NC = 2
NS = 16
NP = 10112
SEG = NP // NS
CH = 128

EPT = E // NS
NFULL = EPT // CH
REM = EPT - NFULL * CH
NCHUNK = NFULL + 1

EPW = E // (NC * NS)
DNF = EPW // CH
DREM = EPW - DNF * CH


def _mlp_body(x_ref, w0_ref, b0_ref, gamma_ref, beta_ref, w1_ref, b1_ref, out_ref):
    x = x_ref[...]
    h = jnp.dot(x, w0_ref[...], preferred_element_type=jnp.float32)
    h = h + b0_ref[...][None, :]
    mu = jnp.mean(h, axis=0, keepdims=True)
    d = h - mu
    var = jnp.mean(d * d, axis=0, keepdims=True)
    h = d / jnp.sqrt(var + BN_EPS) * gamma_ref[...][None, :] + beta_ref[...][None, :]
    h = jnp.maximum(h, 0.0)
    out_ref[...] = (
        jnp.dot(h, w1_ref[...], preferred_element_type=jnp.float32)
        + b1_ref[...][None, :]
    )


_mlp = pl.pallas_call(
    _mlp_body,
    out_shape=jax.ShapeDtypeStruct((N, OUT_C), jnp.float32),
)


def _deg_body(edge_ref, deg_out, idxb, ones, seg, degsp):
    c = lax.axis_index("c")
    s = lax.axis_index("s")
    w = s * NC + c
    ebase = w * EPW

    def fill_ones(i, carry):
        ones[i] = jnp.zeros((16,), jnp.float32) + 1.0
        return carry

    lax.fori_loop(0, CH, fill_ones, 0)

    cval = jnp.where(c == 0, jnp.float32(1.0), jnp.float32(0.0))

    def fill_seg(i, carry):
        seg[i] = jnp.zeros((16,), jnp.float32) + cval
        return carry

    lax.fori_loop(0, SEG, fill_seg, 0)
    pltpu.sync_copy(seg, degsp.at[pl.ds(s * SEG, SEG)])
    plsc.subcore_barrier()

    def dchunk(j, carry):
        pltpu.sync_copy(edge_ref.at[1, pl.ds(ebase + j * CH, CH)], idxb.at[0])
        pltpu.sync_copy(ones, degsp.at[idxb.at[0]], add=True)
        return carry

    lax.fori_loop(0, DNF, dchunk, 0)

    def fill_pad(l, carry):
        idxb[0, pl.ds(l * 16, 16)] = jnp.zeros((16,), jnp.int32) + N
        return carry

    lax.fori_loop(0, CH // 16, fill_pad, 0)
    pltpu.sync_copy(edge_ref.at[1, pl.ds(ebase + DNF * CH, DREM)],
                    idxb.at[0, pl.ds(0, DREM)])
    pltpu.sync_copy(ones, degsp.at[idxb.at[0]], add=True)

    plsc.subcore_barrier()
    pltpu.sync_copy(degsp.at[pl.ds(s * SEG, SEG)], seg)
    pltpu.sync_copy(seg, deg_out.at[c, pl.ds(s * SEG, SEG)])


_deg_kernel = functools.partial(
    pl.kernel,
    out_type=jax.ShapeDtypeStruct((NC, NP, 16), jnp.float32),
    mesh=plsc.VectorSubcoreMesh(core_axis_name="c", subcore_axis_name="s"),
    scratch_types=[
        pltpu.VMEM((1, CH), jnp.int32),
        pltpu.VMEM((CH, 16), jnp.float32),
        pltpu.VMEM((SEG, 16), jnp.float32),
        pltpu.VMEM_SHARED((NP + 16, 16), jnp.float32),
    ],
)(_deg_body)


def _prep_body(deg2_ref, out0_ref, u0_ref, g_ref, coef_ref):
    deg = deg2_ref[0] + deg2_ref[1]
    sinv = 1.0 / jnp.sqrt(deg)
    coef_ref[...] = (1.0 - ALPHA) * sinv * sinv
    s_n = sinv[:N, 0:1]
    u0 = out0_ref[...] * s_n
    g = ALPHA * u0
    zpad = jnp.zeros((NP - N, HALF), jnp.float32)
    u0_ref[pl.ds(0, N), :] = u0[:, :HALF]
    u0_ref[pl.ds(N, NP - N), :] = zpad
    u0_ref[pl.ds(NP, N), :] = u0[:, HALF:]
    u0_ref[pl.ds(NP + N, NP - N), :] = zpad
    g_ref[pl.ds(0, N), :] = g[:, :HALF]
    g_ref[pl.ds(N, NP - N), :] = zpad
    g_ref[pl.ds(NP, N), :] = g[:, HALF:]
    g_ref[pl.ds(NP + N, NP - N), :] = zpad


_prep = pl.pallas_call(
    _prep_body,
    out_shape=[
        jax.ShapeDtypeStruct((2 * NP, HALF), jnp.float32),
        jax.ShapeDtypeStruct((2 * NP, HALF), jnp.float32),
        jax.ShapeDtypeStruct((NP, 16), jnp.float32),
    ],
)


def _appnp_body(edge_ref, u0_ref, coef_ref, g_ref, u_ref,
                src_idx, dst_idx, gbuf, bbuf, gvec, cvec, agg):
    c = lax.axis_index("c")
    t = lax.axis_index("s")
    ebase = t * EPT
    rbase = t * SEG
    cnp = c * NP

    def stage_row(j, carry):
        pltpu.sync_copy(edge_ref.at[0, pl.ds(ebase + j * CH, CH)], src_idx.at[j])
        pltpu.sync_copy(edge_ref.at[1, pl.ds(ebase + j * CH, CH)], dst_idx.at[j])
        return carry

    lax.fori_loop(0, NFULL, stage_row, 0)

    def fill_pad(l, carry):
        src_idx[NFULL, pl.ds(REM + l * 16, 16)] = jnp.zeros((16,), jnp.int32)
        dst_idx[NFULL, pl.ds(REM + l * 16, 16)] = jnp.zeros((16,), jnp.int32) + N
        return carry

    lax.fori_loop(0, (CH - REM) // 16, fill_pad, 0)
    pltpu.sync_copy(edge_ref.at[0, pl.ds(ebase + NFULL * CH, REM)],
                    src_idx.at[NFULL, pl.ds(0, REM)])
    pltpu.sync_copy(edge_ref.at[1, pl.ds(ebase + NFULL * CH, REM)],
                    dst_idx.at[NFULL, pl.ds(0, REM)])

    def add_off(j, carry):
        def add_lane(l, inner):
            v = src_idx[j, pl.ds(l * 16, 16)]
            src_idx[j, pl.ds(l * 16, 16)] = v + cnp
            return inner

        return lax.fori_loop(0, CH // 16, add_lane, carry)

    lax.fori_loop(0, NCHUNK, add_off, 0)

    pltpu.sync_copy(coef_ref.at[pl.ds(rbase, SEG)], cvec)
    pltpu.sync_copy(g_ref.at[pl.ds(cnp + rbase, SEG)], gvec)

    pltpu.sync_copy(u0_ref.at[pl.ds(cnp + rbase, SEG)], bbuf)
    pltpu.sync_copy(bbuf, u_ref.at[pl.ds(cnp + rbase, SEG)])
    pltpu.sync_copy(bbuf, agg.at[pl.ds(rbase, SEG)])
    plsc.subcore_barrier()

    def do_chunk(j, carry):
        pltpu.sync_copy(u_ref.at[src_idx.at[j]], gbuf)
        pltpu.sync_copy(gbuf, agg.at[dst_idx.at[j]], add=True)
        return carry

    def blend_row(i, carry):
        ci = cvec[i, 0]
        bbuf[i, pl.ds(0, 16)] = bbuf[i, pl.ds(0, 16)] * ci + gvec[i, pl.ds(0, 16)]
        bbuf[i, pl.ds(16, 16)] = bbuf[i, pl.ds(16, 16)] * ci + gvec[i, pl.ds(16, 16)]
        return carry

    def round_body(k, carry):
        lax.fori_loop(0, NCHUNK, do_chunk, 0)
        plsc.subcore_barrier()
        pltpu.sync_copy(agg.at[pl.ds(rbase, SEG)], bbuf)
        lax.fori_loop(0, SEG, blend_row, 0)
        pltpu.sync_copy(bbuf, u_ref.at[pl.ds(cnp + rbase, SEG)])
        pltpu.sync_copy(bbuf, agg.at[pl.ds(rbase, SEG)])
        plsc.subcore_barrier()
        return carry

    lax.fori_loop(0, K, round_body, 0)


_appnp = functools.partial(
    pl.kernel,
    out_type=jax.ShapeDtypeStruct((2 * NP, HALF), jnp.float32),
    mesh=plsc.VectorSubcoreMesh(core_axis_name="c", subcore_axis_name="s"),
    scratch_types=[
        pltpu.VMEM((NCHUNK, CH), jnp.int32),
        pltpu.VMEM((NCHUNK, CH), jnp.int32),
        pltpu.VMEM((CH, HALF), jnp.float32),
        pltpu.VMEM((SEG, HALF), jnp.float32),
        pltpu.VMEM((SEG, HALF), jnp.float32),
        pltpu.VMEM((SEG, 16), jnp.float32),
        pltpu.VMEM_SHARED((NP + 16, HALF), jnp.float32),
    ],
)(_appnp_body)


def _final_body(u_ref, deg2_ref, out_ref):
    deg = deg2_ref[0] + deg2_ref[1]
    srt = jnp.sqrt(deg[:N, 0:1])
    u64 = jnp.concatenate([u_ref[pl.ds(0, N), :], u_ref[pl.ds(NP, N), :]], axis=1)
    o = u64 * srt
    m = jnp.max(o, axis=1, keepdims=True)
    e = o - m
    lse = jnp.log(jnp.sum(jnp.exp(e), axis=1, keepdims=True))
    out_ref[...] = e - lse


_final = pl.pallas_call(
    _final_body,
    out_shape=jax.ShapeDtypeStruct((N, OUT_C), jnp.float32),
)


def kernel(x, edge_index, W0, b0, gamma, beta, W1, b1):
    out0 = _mlp(x, W0, b0, gamma, beta, W1, b1)
    deg2 = _deg_kernel(edge_index)
    u0, g, coef = _prep(deg2, out0)
    u = _appnp(edge_index, u0, coef, g)
    return _final(u, deg2)


# speedup vs baseline: 13.2598x; 13.2598x over previous
import functools

import jax
import jax.numpy as jnp
from jax import lax
from jax.experimental import pallas as pl
from jax.experimental.pallas import tpu as pltpu
from jax.experimental.pallas import tpu_sc as plsc

N = 10000
E = 320000
OUT_C = 64
HALF = 32
K = 10
ALPHA = 0.1
BN_EPS = 1e-5

NC = 2
NS = 16
NP = 10112
SEG = NP // NS
CH = 128

EPT = E // NS
NFULL = EPT // CH
REM = EPT - NFULL * CH
NCHUNK = NFULL + 1

EPW = E // (NC * NS)
DNF = EPW // CH
DREM = EPW - DNF * CH


def _mlp_body(x_ref, w0_ref, b0_ref, gamma_ref, beta_ref, w1_ref, b1_ref, out_ref):
    x = x_ref[...]
    h = jnp.dot(x, w0_ref[...], preferred_element_type=jnp.float32)
    h = h + b0_ref[...][None, :]
    mu = jnp.mean(h, axis=0, keepdims=True)
    d = h - mu
    var = jnp.mean(d * d, axis=0, keepdims=True)
    h = d / jnp.sqrt(var + BN_EPS) * gamma_ref[...][None, :] + beta_ref[...][None, :]
    h = jnp.maximum(h, 0.0)
    out_ref[...] = (
        jnp.dot(h, w1_ref[...], preferred_element_type=jnp.float32)
        + b1_ref[...][None, :]
    )


_mlp = pl.pallas_call(
    _mlp_body,
    out_shape=jax.ShapeDtypeStruct((N, OUT_C), jnp.float32),
)


def _deg_body(edge_ref, deg_out, idxb, ones, seg, degsp):
    c = lax.axis_index("c")
    s = lax.axis_index("s")
    w = s * NC + c
    ebase = w * EPW

    def fill_ones(i, carry):
        ones[i] = jnp.zeros((16,), jnp.float32) + 1.0
        return carry

    lax.fori_loop(0, CH, fill_ones, 0)

    def fill_seg0(i, carry):
        seg[i] = jnp.zeros((16,), jnp.float32)
        return carry

    lax.fori_loop(0, SEG, fill_seg0, 0)

    @pl.when(c == 0)
    def _fill_seg1():
        def fill_seg1(i, carry):
            seg[i] = jnp.zeros((16,), jnp.float32) + 1.0
            return carry

        lax.fori_loop(0, SEG, fill_seg1, 0)
    pltpu.sync_copy(seg, degsp.at[pl.ds(s * SEG, SEG)])
    plsc.subcore_barrier()

    def dchunk(j, carry):
        pltpu.sync_copy(edge_ref.at[pl.ds(E + ebase + j * CH, CH)], idxb.at[0])
        pltpu.sync_copy(ones, degsp.at[idxb.at[0]], add=True)
        return carry

    lax.fori_loop(0, DNF, dchunk, 0)

    def fill_pad(l, carry):
        idxb[0, pl.ds(l * 16, 16)] = jnp.zeros((16,), jnp.int32) + N
        return carry

    lax.fori_loop(0, CH // 16, fill_pad, 0)
    pltpu.sync_copy(edge_ref.at[pl.ds(E + ebase + DNF * CH, DREM)],
                    idxb.at[0, pl.ds(0, DREM)])
    pltpu.sync_copy(ones, degsp.at[idxb.at[0]], add=True)

    plsc.subcore_barrier()
    pltpu.sync_copy(degsp.at[pl.ds(s * SEG, SEG)], seg)
    pltpu.sync_copy(seg, deg_out.at[c, pl.ds(s * SEG, SEG)])


_deg_kernel = functools.partial(
    pl.kernel,
    out_type=jax.ShapeDtypeStruct((NC, NP, 16), jnp.float32),
    mesh=plsc.VectorSubcoreMesh(core_axis_name="c", subcore_axis_name="s"),
    compiler_params=pltpu.CompilerParams(use_tc_tiling_on_sc=False),
    scratch_types=[
        pltpu.VMEM((1, CH), jnp.int32),
        pltpu.VMEM((CH, 16), jnp.float32),
        pltpu.VMEM((SEG, 16), jnp.float32),
        pltpu.VMEM_SHARED((NP + 16, 16), jnp.float32),
    ],
)(_deg_body)


def _prep_body(deg2_ref, out0_ref, u0_ref, g_ref, coef_ref):
    deg = deg2_ref[0] + deg2_ref[1]
    sinv = 1.0 / jnp.sqrt(deg)
    coef_ref[...] = (1.0 - ALPHA) * sinv * sinv
    s_n = sinv[:N, 0:1]
    u0 = out0_ref[...] * s_n
    g = ALPHA * u0
    zpad = jnp.zeros((NP - N, HALF), jnp.float32)
    u0_ref[pl.ds(0, N), :] = u0[:, :HALF]
    u0_ref[pl.ds(N, NP - N), :] = zpad
    u0_ref[pl.ds(NP, N), :] = u0[:, HALF:]
    u0_ref[pl.ds(NP + N, NP - N), :] = zpad
    g_ref[pl.ds(0, N), :] = g[:, :HALF]
    g_ref[pl.ds(N, NP - N), :] = zpad
    g_ref[pl.ds(NP, N), :] = g[:, HALF:]
    g_ref[pl.ds(NP + N, NP - N), :] = zpad


_prep = pl.pallas_call(
    _prep_body,
    out_shape=[
        jax.ShapeDtypeStruct((2 * NP, HALF), jnp.float32),
        jax.ShapeDtypeStruct((2 * NP, HALF), jnp.float32),
        jax.ShapeDtypeStruct((NP, 16), jnp.float32),
    ],
)


def _appnp_body(edge_ref, u0_ref, coef_ref, g_ref, u_ref,
                src_idx, dst_idx, gbuf, bbuf, gvec, cvec, agg):
    c = lax.axis_index("c")
    t = lax.axis_index("s")
    ebase = t * EPT
    rbase = t * SEG
    cnp = c * NP

    def stage_row(j, carry):
        pltpu.sync_copy(edge_ref.at[pl.ds(ebase + j * CH, CH)], src_idx.at[j])
        pltpu.sync_copy(edge_ref.at[pl.ds(E + ebase + j * CH, CH)], dst_idx.at[j])
        return carry

    lax.fori_loop(0, NFULL, stage_row, 0)

    def fill_pad(l, carry):
        src_idx[NFULL, pl.ds(REM + l * 16, 16)] = jnp.zeros((16,), jnp.int32)
        dst_idx[NFULL, pl.ds(REM + l * 16, 16)] = jnp.zeros((16,), jnp.int32) + N
        return carry

    lax.fori_loop(0, (CH - REM) // 16, fill_pad, 0)
    pltpu.sync_copy(edge_ref.at[pl.ds(ebase + NFULL * CH, REM)],
                    src_idx.at[NFULL, pl.ds(0, REM)])
    pltpu.sync_copy(edge_ref.at[pl.ds(E + ebase + NFULL * CH, REM)],
                    dst_idx.at[NFULL, pl.ds(0, REM)])

    def add_off(j, carry):
        def add_lane(l, inner):
            v = src_idx[j, pl.ds(l * 16, 16)]
            src_idx[j, pl.ds(l * 16, 16)] = v + cnp
            return inner

        return lax.fori_loop(0, CH // 16, add_lane, carry)

    lax.fori_loop(0, NCHUNK, add_off, 0)

    pltpu.sync_copy(coef_ref.at[pl.ds(rbase, SEG)], cvec)
    pltpu.sync_copy(g_ref.at[pl.ds(cnp + rbase, SEG)], gvec)

    pltpu.sync_copy(u0_ref.at[pl.ds(cnp + rbase, SEG)], bbuf)
    pltpu.sync_copy(bbuf, u_ref.at[pl.ds(cnp + rbase, SEG)])
    pltpu.sync_copy(bbuf, agg.at[pl.ds(rbase, SEG)])
    plsc.subcore_barrier()

    def do_chunk(j, carry):
        pltpu.sync_copy(u_ref.at[src_idx.at[j]], gbuf)
        pltpu.sync_copy(gbuf, agg.at[dst_idx.at[j]], add=True)
        return carry

    def blend_row(i, carry):
        ci = cvec[i][0]
        bbuf[i, pl.ds(0, 16)] = bbuf[i, pl.ds(0, 16)] * ci + gvec[i, pl.ds(0, 16)]
        bbuf[i, pl.ds(16, 16)] = bbuf[i, pl.ds(16, 16)] * ci + gvec[i, pl.ds(16, 16)]
        return carry

    def round_body(k, carry):
        lax.fori_loop(0, NCHUNK, do_chunk, 0)
        plsc.subcore_barrier()
        pltpu.sync_copy(agg.at[pl.ds(rbase, SEG)], bbuf)
        lax.fori_loop(0, SEG, blend_row, 0)
        pltpu.sync_copy(bbuf, u_ref.at[pl.ds(cnp + rbase, SEG)])
        pltpu.sync_copy(bbuf, agg.at[pl.ds(rbase, SEG)])
        plsc.subcore_barrier()
        return carry

    lax.fori_loop(0, K, round_body, 0)


_appnp = functools.partial(
    pl.kernel,
    out_type=jax.ShapeDtypeStruct((2 * NP, HALF), jnp.float32),
    mesh=plsc.VectorSubcoreMesh(core_axis_name="c", subcore_axis_name="s"),
    compiler_params=pltpu.CompilerParams(use_tc_tiling_on_sc=False),
    scratch_types=[
        pltpu.VMEM((NCHUNK, CH), jnp.int32),
        pltpu.VMEM((NCHUNK, CH), jnp.int32),
        pltpu.VMEM((CH, HALF), jnp.float32),
        pltpu.VMEM((SEG, HALF), jnp.float32),
        pltpu.VMEM((SEG, HALF), jnp.float32),
        pltpu.VMEM((SEG, 16), jnp.float32),
        pltpu.VMEM_SHARED((NP + 16, HALF), jnp.float32),
    ],
)(_appnp_body)


def _final_body(u_ref, deg2_ref, out_ref):
    deg = deg2_ref[0] + deg2_ref[1]
    srt = jnp.sqrt(deg[:N, 0:1])
    u64 = jnp.concatenate([u_ref[pl.ds(0, N), :], u_ref[pl.ds(NP, N), :]], axis=1)
    o = u64 * srt
    m = jnp.max(o, axis=1, keepdims=True)
    e = o - m
    lse = jnp.log(jnp.sum(jnp.exp(e), axis=1, keepdims=True))
    out_ref[...] = e - lse


_final = pl.pallas_call(
    _final_body,
    out_shape=jax.ShapeDtypeStruct((N, OUT_C), jnp.float32),
)


def kernel(x, edge_index, W0, b0, gamma, beta, W1, b1):
    edge_flat = edge_index.reshape(2 * E)
    out0 = _mlp(x, W0, b0, gamma, beta, W1, b1)
    deg2 = _deg_kernel(edge_flat)
    u0, g, coef = _prep(deg2, out0)
    u = _appnp(edge_flat, u0, coef, g)
    return _final(u, deg2)


# speedup vs baseline: 29.8957x; 2.2546x over previous
import functools

import jax
import jax.numpy as jnp
from jax import lax
from jax.experimental import pallas as pl
from jax.experimental.pallas import tpu as pltpu
from jax.experimental.pallas import tpu_sc as plsc

N = 10000
E = 320000
OUT_C = 64
HALF = 32
K = 10
ALPHA = 0.1
BN_EPS = 1e-5

NC = 2
NS = 16
NP = 10112
SEG = NP // NS
CH = 128

EPT = E // NS
C2 = 400
NCH2 = EPT // C2
SB = 158
NSB = SEG // SB

EPW = E // (NC * NS)
DNF = EPW // CH
DREM = EPW - DNF * CH


def _mlp_body(x_ref, w0_ref, b0_ref, gamma_ref, beta_ref, w1_ref, b1_ref, out_ref):
    x = x_ref[...]
    h = jnp.dot(x, w0_ref[...], preferred_element_type=jnp.float32)
    h = h + b0_ref[...][None, :]
    mu = jnp.mean(h, axis=0, keepdims=True)
    d = h - mu
    var = jnp.mean(d * d, axis=0, keepdims=True)
    h = d / jnp.sqrt(var + BN_EPS) * gamma_ref[...][None, :] + beta_ref[...][None, :]
    h = jnp.maximum(h, 0.0)
    out_ref[...] = (
        jnp.dot(h, w1_ref[...], preferred_element_type=jnp.float32)
        + b1_ref[...][None, :]
    )


_mlp = pl.pallas_call(
    _mlp_body,
    out_shape=jax.ShapeDtypeStruct((N, OUT_C), jnp.float32),
)


def _deg_body(edge_ref, deg_out, idxb, ones, seg, degsp):
    c = lax.axis_index("c")
    s = lax.axis_index("s")
    w = s * NC + c
    ebase = w * EPW

    def fill_ones(i, carry):
        ones[i] = jnp.zeros((16,), jnp.float32) + 1.0
        return carry

    lax.fori_loop(0, CH, fill_ones, 0)

    def fill_seg0(i, carry):
        seg[i] = jnp.zeros((16,), jnp.float32)
        return carry

    lax.fori_loop(0, SEG, fill_seg0, 0)

    @pl.when(c == 0)
    def _fill_seg1():
        def fill_seg1(i, carry):
            seg[i] = jnp.zeros((16,), jnp.float32) + 1.0
            return carry

        lax.fori_loop(0, SEG, fill_seg1, 0)
    pltpu.sync_copy(seg, degsp.at[pl.ds(s * SEG, SEG)])
    plsc.subcore_barrier()

    def dchunk(j, carry):
        pltpu.sync_copy(edge_ref.at[pl.ds(E + ebase + j * CH, CH)], idxb.at[0])
        pltpu.sync_copy(ones, degsp.at[idxb.at[0]], add=True)
        return carry

    lax.fori_loop(0, DNF, dchunk, 0)

    def fill_pad(l, carry):
        idxb[0, pl.ds(l * 16, 16)] = jnp.zeros((16,), jnp.int32) + N
        return carry

    lax.fori_loop(0, CH // 16, fill_pad, 0)
    pltpu.sync_copy(edge_ref.at[pl.ds(E + ebase + DNF * CH, DREM)],
                    idxb.at[0, pl.ds(0, DREM)])
    pltpu.sync_copy(ones, degsp.at[idxb.at[0]], add=True)

    plsc.subcore_barrier()
    pltpu.sync_copy(degsp.at[pl.ds(s * SEG, SEG)], seg)
    pltpu.sync_copy(seg, deg_out.at[c, pl.ds(s * SEG, SEG)])


_deg_kernel = functools.partial(
    pl.kernel,
    out_type=jax.ShapeDtypeStruct((NC, NP, 16), jnp.float32),
    mesh=plsc.VectorSubcoreMesh(core_axis_name="c", subcore_axis_name="s"),
    compiler_params=pltpu.CompilerParams(use_tc_tiling_on_sc=False),
    scratch_types=[
        pltpu.VMEM((1, CH), jnp.int32),
        pltpu.VMEM((CH, 16), jnp.float32),
        pltpu.VMEM((SEG, 16), jnp.float32),
        pltpu.VMEM_SHARED((NP + 16, 16), jnp.float32),
    ],
)(_deg_body)


def _prep_body(deg2_ref, out0_ref, u0_ref, g_ref, coef_ref):
    deg = deg2_ref[0] + deg2_ref[1]
    sinv = 1.0 / jnp.sqrt(deg)
    coef_ref[...] = (1.0 - ALPHA) * sinv * sinv
    s_n = sinv[:N, 0:1]
    u0 = out0_ref[...] * s_n
    g = ALPHA * u0
    zpad = jnp.zeros((NP - N, HALF), jnp.float32)
    u0_ref[pl.ds(0, N), :] = u0[:, :HALF]
    u0_ref[pl.ds(N, NP - N), :] = zpad
    u0_ref[pl.ds(NP, N), :] = u0[:, HALF:]
    u0_ref[pl.ds(NP + N, NP - N), :] = zpad
    g_ref[pl.ds(0, N), :] = g[:, :HALF]
    g_ref[pl.ds(N, NP - N), :] = zpad
    g_ref[pl.ds(NP, N), :] = g[:, HALF:]
    g_ref[pl.ds(NP + N, NP - N), :] = zpad


_prep = pl.pallas_call(
    _prep_body,
    out_shape=[
        jax.ShapeDtypeStruct((2 * NP, HALF), jnp.float32),
        jax.ShapeDtypeStruct((2 * NP, HALF), jnp.float32),
        jax.ShapeDtypeStruct((NP, 16), jnp.float32),
    ],
)


def _appnp_body(edge_ref, u0_ref, coef_ref, g_ref, u_ref,
                src_idx, dst_idx, gbuf0, gbuf1, bbuf, gvec, cvec, agg,
                sg0, sg1):
    c = lax.axis_index("c")
    t = lax.axis_index("s")
    ebase = t * EPT
    rbase = t * SEG
    cnp = c * NP

    def stage_row(j, carry):
        pltpu.sync_copy(edge_ref.at[pl.ds(ebase + j * C2, C2)], src_idx.at[j])
        pltpu.sync_copy(edge_ref.at[pl.ds(E + ebase + j * C2, C2)], dst_idx.at[j])
        return carry

    lax.fori_loop(0, NCH2, stage_row, 0)

    def add_off(j, carry):
        def add_lane(l, inner):
            v = src_idx[j, pl.ds(l * 16, 16)]
            src_idx[j, pl.ds(l * 16, 16)] = v + cnp
            return inner

        return lax.fori_loop(0, C2 // 16, add_lane, carry)

    lax.fori_loop(0, NCH2, add_off, 0)

    pltpu.sync_copy(coef_ref.at[pl.ds(rbase, SEG)], cvec)
    pltpu.sync_copy(g_ref.at[pl.ds(cnp + rbase, SEG)], gvec)

    def init_sub(sb, carry):
        off = rbase + sb * SB
        pltpu.sync_copy(u0_ref.at[pl.ds(cnp + off, SB)], bbuf)
        pltpu.sync_copy(bbuf, u_ref.at[pl.ds(cnp + off, SB)])
        pltpu.sync_copy(bbuf, agg.at[pl.ds(off, SB)])
        return carry

    lax.fori_loop(0, NSB, init_sub, 0)
    plsc.subcore_barrier()

    bufs = (gbuf0, gbuf1)
    sems = (sg0, sg1)

    def gather_issue(j, slot):
        pltpu.async_copy(u_ref.at[src_idx.at[j]], bufs[slot], sems[slot])

    def gather_wait(slot):
        pltpu.make_async_copy(u_ref.at[pl.ds(0, C2)], bufs[slot], sems[slot]).wait()

    def scatter_sync(j, slot):
        pltpu.sync_copy(bufs[slot], agg.at[dst_idx.at[j]], add=True)

    def round_body(k, carry):
        gather_issue(0, 0)

        def pair(gp, c2):
            j0 = 2 * gp
            j1 = 2 * gp + 1
            gather_wait(0)
            gather_issue(j1, 1)
            scatter_sync(j0, 0)
            gather_wait(1)

            @pl.when(j1 + 1 < NCH2)
            def _():
                gather_issue(j1 + 1, 0)

            scatter_sync(j1, 1)
            return c2

        lax.fori_loop(0, NCH2 // 2, pair, 0)
        plsc.subcore_barrier()

        def blend_sub(sb, carry2):
            off = rbase + sb * SB
            pltpu.sync_copy(agg.at[pl.ds(off, SB)], bbuf)

            def blend_row(i, c3):
                ci = cvec[sb * SB + i][0]
                bbuf[i, pl.ds(0, 16)] = (
                    bbuf[i, pl.ds(0, 16)] * ci + gvec[sb * SB + i, pl.ds(0, 16)]
                )
                bbuf[i, pl.ds(16, 16)] = (
                    bbuf[i, pl.ds(16, 16)] * ci + gvec[sb * SB + i, pl.ds(16, 16)]
                )
                return c3

            lax.fori_loop(0, SB, blend_row, 0)
            pltpu.sync_copy(bbuf, u_ref.at[pl.ds(cnp + off, SB)])
            pltpu.sync_copy(bbuf, agg.at[pl.ds(off, SB)])
            return carry2

        lax.fori_loop(0, NSB, blend_sub, 0)
        plsc.subcore_barrier()
        return carry

    lax.fori_loop(0, K, round_body, 0)


_appnp = functools.partial(
    pl.kernel,
    out_type=jax.ShapeDtypeStruct((2 * NP, HALF), jnp.float32),
    mesh=plsc.VectorSubcoreMesh(core_axis_name="c", subcore_axis_name="s"),
    compiler_params=pltpu.CompilerParams(use_tc_tiling_on_sc=False),
    scratch_types=[
        pltpu.VMEM((NCH2, C2), jnp.int32),
        pltpu.VMEM((NCH2, C2), jnp.int32),
        pltpu.VMEM((C2, HALF), jnp.float32),
        pltpu.VMEM((C2, HALF), jnp.float32),
        pltpu.VMEM((SB, HALF), jnp.float32),
        pltpu.VMEM((SEG, HALF), jnp.float32),
        pltpu.VMEM((SEG, 16), jnp.float32),
        pltpu.VMEM_SHARED((NP + 16, HALF), jnp.float32),
        pltpu.SemaphoreType.DMA,
        pltpu.SemaphoreType.DMA,
    ],
)(_appnp_body)


def _final_body(u_ref, deg2_ref, out_ref):
    deg = deg2_ref[0] + deg2_ref[1]
    srt = jnp.sqrt(deg[:N, 0:1])
    u64 = jnp.concatenate([u_ref[pl.ds(0, N), :], u_ref[pl.ds(NP, N), :]], axis=1)
    o = u64 * srt
    m = jnp.max(o, axis=1, keepdims=True)
    e = o - m
    lse = jnp.log(jnp.sum(jnp.exp(e), axis=1, keepdims=True))
    out_ref[...] = e - lse


_final = pl.pallas_call(
    _final_body,
    out_shape=jax.ShapeDtypeStruct((N, OUT_C), jnp.float32),
)


def kernel(x, edge_index, W0, b0, gamma, beta, W1, b1):
    edge_flat = edge_index.reshape(2 * E)
    out0 = _mlp(x, W0, b0, gamma, beta, W1, b1)
    deg2 = _deg_kernel(edge_flat)
    u0, g, coef = _prep(deg2, out0)
    u = _appnp(edge_flat, u0, coef, g)
    return _final(u, deg2)


# speedup vs baseline: 32.0085x; 1.0707x over previous
import functools

import jax
import jax.numpy as jnp
from jax import lax
from jax.experimental import pallas as pl
from jax.experimental.pallas import tpu as pltpu
from jax.experimental.pallas import tpu_sc as plsc

N = 10000
E = 320000
OUT_C = 64
HALF = 32
K = 10
ALPHA = 0.1
BN_EPS = 1e-5

NC = 2
NS = 16
NP = 10112
SEG = NP // NS
CH = 128

EPT = E // NS
C2 = 200
NCH2 = EPT // C2
NSLOT = 4
NG = NCH2 // NSLOT
SB = 79
NSB = SEG // SB

EPW = E // (NC * NS)
DNF = EPW // CH
DREM = EPW - DNF * CH


def _mlp_body(x_ref, w0_ref, b0_ref, gamma_ref, beta_ref, w1_ref, b1_ref, out_ref):
    x = x_ref[...]
    h = jnp.dot(x, w0_ref[...], preferred_element_type=jnp.float32)
    h = h + b0_ref[...][None, :]
    mu = jnp.mean(h, axis=0, keepdims=True)
    d = h - mu
    var = jnp.mean(d * d, axis=0, keepdims=True)
    h = d / jnp.sqrt(var + BN_EPS) * gamma_ref[...][None, :] + beta_ref[...][None, :]
    h = jnp.maximum(h, 0.0)
    out_ref[...] = (
        jnp.dot(h, w1_ref[...], preferred_element_type=jnp.float32)
        + b1_ref[...][None, :]
    )


_mlp = pl.pallas_call(
    _mlp_body,
    out_shape=jax.ShapeDtypeStruct((N, OUT_C), jnp.float32),
)


def _deg_body(edge_ref, deg_out, idxb, ones, seg, degsp):
    c = lax.axis_index("c")
    s = lax.axis_index("s")
    w = s * NC + c
    ebase = w * EPW

    def fill_ones(i, carry):
        ones[i] = jnp.zeros((16,), jnp.float32) + 1.0
        return carry

    lax.fori_loop(0, CH, fill_ones, 0)

    def fill_seg0(i, carry):
        seg[i] = jnp.zeros((16,), jnp.float32)
        return carry

    lax.fori_loop(0, SEG, fill_seg0, 0)

    @pl.when(c == 0)
    def _fill_seg1():
        def fill_seg1(i, carry):
            seg[i] = jnp.zeros((16,), jnp.float32) + 1.0
            return carry

        lax.fori_loop(0, SEG, fill_seg1, 0)
    pltpu.sync_copy(seg, degsp.at[pl.ds(s * SEG, SEG)])
    plsc.subcore_barrier()

    def dchunk(j, carry):
        pltpu.sync_copy(edge_ref.at[pl.ds(E + ebase + j * CH, CH)], idxb.at[0])
        pltpu.sync_copy(ones, degsp.at[idxb.at[0]], add=True)
        return carry

    lax.fori_loop(0, DNF, dchunk, 0)

    def fill_pad(l, carry):
        idxb[0, pl.ds(l * 16, 16)] = jnp.zeros((16,), jnp.int32) + N
        return carry

    lax.fori_loop(0, CH // 16, fill_pad, 0)
    pltpu.sync_copy(edge_ref.at[pl.ds(E + ebase + DNF * CH, DREM)],
                    idxb.at[0, pl.ds(0, DREM)])
    pltpu.sync_copy(ones, degsp.at[idxb.at[0]], add=True)

    plsc.subcore_barrier()
    pltpu.sync_copy(degsp.at[pl.ds(s * SEG, SEG)], seg)
    pltpu.sync_copy(seg, deg_out.at[c, pl.ds(s * SEG, SEG)])


_deg_kernel = functools.partial(
    pl.kernel,
    out_type=jax.ShapeDtypeStruct((NC, NP, 16), jnp.float32),
    mesh=plsc.VectorSubcoreMesh(core_axis_name="c", subcore_axis_name="s"),
    compiler_params=pltpu.CompilerParams(use_tc_tiling_on_sc=False),
    scratch_types=[
        pltpu.VMEM((1, CH), jnp.int32),
        pltpu.VMEM((CH, 16), jnp.float32),
        pltpu.VMEM((SEG, 16), jnp.float32),
        pltpu.VMEM_SHARED((NP + 16, 16), jnp.float32),
    ],
)(_deg_body)


def _prep_body(deg2_ref, out0_ref, u0_ref, g_ref, coef_ref):
    deg = deg2_ref[0] + deg2_ref[1]
    sinv = 1.0 / jnp.sqrt(deg)
    coef_ref[...] = (1.0 - ALPHA) * sinv * sinv
    s_n = sinv[:N, 0:1]
    u0 = out0_ref[...] * s_n
    g = ALPHA * u0
    zpad = jnp.zeros((NP - N, HALF), jnp.float32)
    u0_ref[pl.ds(0, N), :] = u0[:, :HALF]
    u0_ref[pl.ds(N, NP - N), :] = zpad
    u0_ref[pl.ds(NP, N), :] = u0[:, HALF:]
    u0_ref[pl.ds(NP + N, NP - N), :] = zpad
    g_ref[pl.ds(0, N), :] = g[:, :HALF]
    g_ref[pl.ds(N, NP - N), :] = zpad
    g_ref[pl.ds(NP, N), :] = g[:, HALF:]
    g_ref[pl.ds(NP + N, NP - N), :] = zpad


_prep = pl.pallas_call(
    _prep_body,
    out_shape=[
        jax.ShapeDtypeStruct((2 * NP, HALF), jnp.float32),
        jax.ShapeDtypeStruct((2 * NP, HALF), jnp.float32),
        jax.ShapeDtypeStruct((NP, 16), jnp.float32),
    ],
)


def _appnp_body(edge_ref, u0_ref, coef_ref, g_ref, u_ref,
                src_idx, dst_idx, gbuf0, gbuf1, gbuf2, gbuf3, bbuf, gvec,
                cvec, agg, sg, ss):
    c = lax.axis_index("c")
    t = lax.axis_index("s")
    ebase = t * EPT
    rbase = t * SEG
    cnp = c * NP

    def stage_row(j, carry):
        pltpu.sync_copy(edge_ref.at[pl.ds(ebase + j * C2, C2)], src_idx.at[j])
        pltpu.sync_copy(edge_ref.at[pl.ds(E + ebase + j * C2, C2)], dst_idx.at[j])
        return carry

    lax.fori_loop(0, NCH2, stage_row, 0)

    def add_off(j, carry):
        def add_lane(l, inner):
            v = src_idx[j, pl.ds(l * 16, 16)]
            src_idx[j, pl.ds(l * 16, 16)] = v + cnp
            return inner

        return lax.fori_loop(0, C2 // 16, add_lane, carry)

    lax.fori_loop(0, NCH2, add_off, 0)

    pltpu.sync_copy(coef_ref.at[pl.ds(rbase, SEG)], cvec)
    pltpu.sync_copy(g_ref.at[pl.ds(cnp + rbase, SEG)], gvec)

    def init_sub(sb, carry):
        off = rbase + sb * SB
        pltpu.sync_copy(u0_ref.at[pl.ds(cnp + off, SB)], bbuf)
        pltpu.sync_copy(bbuf, u_ref.at[pl.ds(cnp + off, SB)])
        pltpu.sync_copy(bbuf, agg.at[pl.ds(off, SB)])
        return carry

    lax.fori_loop(0, NSB, init_sub, 0)
    plsc.subcore_barrier()

    bufs = (gbuf0, gbuf1, gbuf2, gbuf3)

    def gather_issue(j, b):
        pltpu.async_copy(u_ref.at[src_idx.at[j]], bufs[b], sg.at[b])

    def gather_wait(b):
        pltpu.make_async_copy(u_ref.at[pl.ds(0, C2)], bufs[b], sg.at[b]).wait()

    def scatter_issue(j, b):
        pltpu.async_copy(bufs[b], agg.at[dst_idx.at[j]], ss.at[b], add=True)

    def scatter_wait(b):
        pltpu.make_async_copy(bufs[b], agg.at[pl.ds(0, C2)], ss.at[b]).wait()

    def blend_row_at(i, base):
        ci = cvec[base + i][0]
        bbuf[i, pl.ds(0, 16)] = (
            bbuf[i, pl.ds(0, 16)] * ci + gvec[base + i, pl.ds(0, 16)]
        )
        bbuf[i, pl.ds(16, 16)] = (
            bbuf[i, pl.ds(16, 16)] * ci + gvec[base + i, pl.ds(16, 16)]
        )

    def round_body(k, carry):
        for b in range(NSLOT):
            gather_issue(b, b)

        def group(g, c2):
            for b in range(NSLOT):
                j = NSLOT * g + b
                gather_wait(b)
                scatter_issue(j, b)

            @pl.when(g + 1 < NG)
            def _():
                for b in range(NSLOT):
                    scatter_wait(b)
                    gather_issue(NSLOT * (g + 1) + b, b)

            return c2

        lax.fori_loop(0, NG, group, 0)
        for b in range(NSLOT):
            scatter_wait(b)
        plsc.subcore_barrier()

        def blend_sub(sb, carry2):
            off = rbase + sb * SB
            pltpu.sync_copy(agg.at[pl.ds(off, SB)], bbuf)

            def blend_row(i, c3):
                blend_row_at(i, sb * SB)
                return c3

            lax.fori_loop(0, SB, blend_row, 0)
            pltpu.sync_copy(bbuf, u_ref.at[pl.ds(cnp + off, SB)])
            pltpu.sync_copy(bbuf, agg.at[pl.ds(off, SB)])
            return carry2

        lax.fori_loop(0, NSB, blend_sub, 0)
        plsc.subcore_barrier()
        return carry

    lax.fori_loop(0, K, round_body, 0)


_appnp = functools.partial(
    pl.kernel,
    out_type=jax.ShapeDtypeStruct((2 * NP, HALF), jnp.float32),
    mesh=plsc.VectorSubcoreMesh(core_axis_name="c", subcore_axis_name="s"),
    compiler_params=pltpu.CompilerParams(use_tc_tiling_on_sc=False),
    scratch_types=[
        pltpu.VMEM((NCH2, C2), jnp.int32),
        pltpu.VMEM((NCH2, C2), jnp.int32),
        pltpu.VMEM((C2, HALF), jnp.float32),
        pltpu.VMEM((C2, HALF), jnp.float32),
        pltpu.VMEM((C2, HALF), jnp.float32),
        pltpu.VMEM((C2, HALF), jnp.float32),
        pltpu.VMEM((SB, HALF), jnp.float32),
        pltpu.VMEM((SEG, HALF), jnp.float32),
        pltpu.VMEM((SEG, 16), jnp.float32),
        pltpu.VMEM_SHARED((NP + 16, HALF), jnp.float32),
        pltpu.SemaphoreType.DMA((NSLOT,)),
        pltpu.SemaphoreType.DMA((NSLOT,)),
    ],
)(_appnp_body)


def _final_body(u_ref, deg2_ref, out_ref):
    deg = deg2_ref[0] + deg2_ref[1]
    srt = jnp.sqrt(deg[:N, 0:1])
    u64 = jnp.concatenate([u_ref[pl.ds(0, N), :], u_ref[pl.ds(NP, N), :]], axis=1)
    o = u64 * srt
    m = jnp.max(o, axis=1, keepdims=True)
    e = o - m
    lse = jnp.log(jnp.sum(jnp.exp(e), axis=1, keepdims=True))
    out_ref[...] = e - lse


_final = pl.pallas_call(
    _final_body,
    out_shape=jax.ShapeDtypeStruct((N, OUT_C), jnp.float32),
)


def kernel(x, edge_index, W0, b0, gamma, beta, W1, b1):
    edge_flat = edge_index.reshape(2 * E)
    out0 = _mlp(x, W0, b0, gamma, beta, W1, b1)
    deg2 = _deg_kernel(edge_flat)
    u0, g, coef = _prep(deg2, out0)
    u = _appnp(edge_flat, u0, coef, g)
    return _final(u, deg2)
